# Initial kernel scaffold; baseline (speedup 1.0000x reference)
import functools

import jax
import jax.numpy as jnp
from jax import lax
from jax.experimental import pallas as pl
from jax.experimental.pallas import tpu as pltpu
from jax.experimental.pallas import tpu_sc as plsc

N_USERS = 6000
D = 128
NC = 2
NS = 16
L = 16
NW = NC * NS
CHUNK = 128
U = 16
BN = 1024


def _mm2_body(x_ref, wl_ref, wr_ref, xl_ref, xr_ref):
  x = x_ref[...]
  xl_ref[...] = jnp.dot(x, wl_ref[...], preferred_element_type=jnp.float32)
  xr_ref[...] = jnp.dot(x, wr_ref[...], preferred_element_type=jnp.float32)


def _mm2(x, wl, wr):
  n = x.shape[0]
  return pl.pallas_call(
      _mm2_body,
      grid=(n // BN,),
      in_specs=[
          pl.BlockSpec((BN, D), lambda i: (i, 0)),
          pl.BlockSpec((D, D), lambda i: (0, 0)),
          pl.BlockSpec((D, D), lambda i: (0, 0)),
      ],
      out_specs=[
          pl.BlockSpec((BN, D), lambda i: (i, 0)),
          pl.BlockSpec((BN, D), lambda i: (i, 0)),
      ],
      out_shape=[jax.ShapeDtypeStruct((n, D), jnp.float32)] * 2,
  )(x, wl, wr)


def _comb_mm2_body(acc_ref, s_ref, b_ref, res_ref, wl_ref, wr_ref,
                   h_ref, xl_ref, xr_ref):
  s = jnp.sum(s_ref[...], axis=0) + 1e-16
  h = (acc_ref[0] + acc_ref[1]) / s[:, None] + b_ref[...] + res_ref[...]
  h = jnp.maximum(h, 0.0)
  h_ref[...] = h
  xl_ref[...] = jnp.dot(h, wl_ref[...], preferred_element_type=jnp.float32)
  xr_ref[...] = jnp.dot(h, wr_ref[...], preferred_element_type=jnp.float32)


def _comb_mm2(acc, s, b, res, wl, wr):
  n = res.shape[0]
  nw = s.shape[0]
  return pl.pallas_call(
      _comb_mm2_body,
      grid=(n // BN,),
      in_specs=[
          pl.BlockSpec((NC, BN, D), lambda i: (0, i, 0)),
          pl.BlockSpec((nw, BN), lambda i: (0, i)),
          pl.BlockSpec((1, D), lambda i: (0, 0)),
          pl.BlockSpec((BN, D), lambda i: (i, 0)),
          pl.BlockSpec((D, D), lambda i: (0, 0)),
          pl.BlockSpec((D, D), lambda i: (0, 0)),
      ],
      out_specs=[
          pl.BlockSpec((BN, D), lambda i: (i, 0)),
          pl.BlockSpec((BN, D), lambda i: (i, 0)),
          pl.BlockSpec((BN, D), lambda i: (i, 0)),
      ],
      out_shape=[jax.ShapeDtypeStruct((n, D), jnp.float32)] * 3,
  )(acc, s, b, res, wl, wr)


def _final_body(acc_ref, s_ref, b_ref, res_ref, y_ref):
  s = jnp.sum(s_ref[...], axis=0) + 1e-16
  y_ref[...] = (acc_ref[0] + acc_ref[1]) / s[:, None] + b_ref[...] + res_ref[...]


def _final(acc, s, b, res):
  n = res.shape[0]
  nw = s.shape[0]
  return pl.pallas_call(
      _final_body,
      grid=(n // BN,),
      in_specs=[
          pl.BlockSpec((NC, BN, D), lambda i: (0, i, 0)),
          pl.BlockSpec((nw, BN), lambda i: (0, i)),
          pl.BlockSpec((1, D), lambda i: (0, 0)),
          pl.BlockSpec((BN, D), lambda i: (i, 0)),
      ],
      out_specs=pl.BlockSpec((BN, D), lambda i: (i, 0)),
      out_shape=jax.ShapeDtypeStruct((n, D), jnp.float32),
  )(acc, s, b, res)


def _sc_mesh():
  return plsc.VectorSubcoreMesh(
      core_axis_name="c", subcore_axis_name="s", num_cores=NC, num_subcores=NS)


@functools.lru_cache(maxsize=None)
def _make_sc_logits(e_pad, n_pad, per_w):
  n_chunks = per_w // CHUNK

  @functools.partial(
      pl.kernel,
      out_type=[
          jax.ShapeDtypeStruct((e_pad,), jnp.float32),
          jax.ShapeDtypeStruct((NW, n_pad), jnp.float32),
      ],
      mesh=_sc_mesh(),
      scratch_types=[
          pltpu.VMEM((CHUNK,), jnp.int32),
          pltpu.VMEM((CHUNK,), jnp.int32),
          pltpu.VMEM((CHUNK, D), jnp.float32),
          pltpu.VMEM((CHUNK, D), jnp.float32),
          pltpu.VMEM((CHUNK,), jnp.float32),
          pltpu.VMEM((D,), jnp.float32),
          pltpu.VMEM((n_pad,), jnp.float32),
          pltpu.SemaphoreType.DMA,
          pltpu.SemaphoreType.DMA,
      ],
  )
  def sc_logits(xl_hbm, xr_hbm, src_hbm, dst_hbm, att_hbm,
                p_hbm, s_hbm,
                src_v, dst_v, xlr, xrr, p_v, att_v, s_v, sem0, sem1):
    cid = lax.axis_index("c")
    sid = lax.axis_index("s")
    wid = cid * NS + sid
    pltpu.sync_copy(att_hbm, att_v)

    def zero_body(i, carry):
      s_v[pl.ds(i * L, L)] = jnp.zeros((L,), jnp.float32)
      return carry

    lax.fori_loop(0, n_pad // L, zero_body, 0)
    row16 = lax.iota(jnp.int32, L)

    def chunk_body(i, carry):
      base = wid * per_w + i * CHUNK
      pltpu.sync_copy(src_hbm.at[pl.ds(base, CHUNK)], src_v)
      pltpu.sync_copy(dst_hbm.at[pl.ds(base, CHUNK)], dst_v)
      cl = pltpu.async_copy(xl_hbm.at[src_v], xlr, sem0)
      cr = pltpu.async_copy(xr_hbm.at[dst_v], xrr, sem1)
      cl.wait()
      cr.wait()
      for g in range(CHUNK // L):
        rows = row16 + (g * L)

        def d_body(dd, acc):
          for u in range(U):
            d = dd * U + u
            col = jnp.full((L,), d, jnp.int32)
            a = plsc.load_gather(xlr, [rows, col])
            b = plsc.load_gather(xrr, [rows, col])
            t = a + b
            t = jnp.maximum(t, 0.2 * t)
            acc = acc + att_v[d] * t
          return acc

        acc = lax.fori_loop(0, D // U, d_body, jnp.zeros((L,), jnp.float32))
        p16 = jnp.exp(acc)
        p_v[pl.ds(g * L, L)] = p16
        plsc.addupdate_scatter(s_v, [dst_v[pl.ds(g * L, L)]], p16)
      pltpu.sync_copy(p_v, p_hbm.at[pl.ds(base, CHUNK)])
      return carry

    lax.fori_loop(0, n_chunks, chunk_body, 0)
    pltpu.sync_copy(s_v, s_hbm.at[wid])

  return sc_logits


@functools.lru_cache(maxsize=None)
def _make_sc_agg(e_pad, n_pad, per_w):
  n_chunks = per_w // CHUNK
  rpt = n_pad // NS

  @functools.partial(
      pl.kernel,
      out_type=jax.ShapeDtypeStruct((NC, n_pad, D), jnp.float32),
      mesh=_sc_mesh(),
      scratch_types=[
          pltpu.VMEM((CHUNK,), jnp.int32),
          pltpu.VMEM((CHUNK,), jnp.int32),
          pltpu.VMEM((CHUNK, D), jnp.float32),
          pltpu.VMEM((CHUNK,), jnp.float32),
          pltpu.VMEM_SHARED((n_pad, D), jnp.float32),
          pltpu.SemaphoreType.DMA,
      ],
  )
  def sc_agg(xl_hbm, src_hbm, dst_hbm, p_hbm, zero_nd_hbm,
             out_hbm,
             src_v, dst_v, rows_v, p_v, acc_sh, sem0):
    cid = lax.axis_index("c")
    sid = lax.axis_index("s")
    wid = cid * NS + sid
    pltpu.sync_copy(zero_nd_hbm.at[pl.ds(sid * rpt, rpt)],
                    acc_sh.at[pl.ds(sid * rpt, rpt)])
    plsc.subcore_barrier()

    def chunk_body(i, carry):
      base = wid * per_w + i * CHUNK
      pltpu.sync_copy(src_hbm.at[pl.ds(base, CHUNK)], src_v)
      pltpu.sync_copy(dst_hbm.at[pl.ds(base, CHUNK)], dst_v)
      pltpu.sync_copy(p_hbm.at[pl.ds(base, CHUNK)], p_v)
      pltpu.async_copy(xl_hbm.at[src_v], rows_v, sem0).wait()

      def scale_body(j, carry2):
        w = p_v[j]
        for q in range(D // L):
          sl = pl.ds(q * L, L)
          rows_v[j, sl] = rows_v[j, sl] * w
        return carry2

      lax.fori_loop(0, CHUNK, scale_body, 0)
      pltpu.sync_copy(rows_v, acc_sh.at[dst_v], add=True)
      return carry

    lax.fori_loop(0, n_chunks, chunk_body, 0)
    plsc.subcore_barrier()
    pltpu.sync_copy(acc_sh.at[pl.ds(sid * rpt, rpt)],
                    out_hbm.at[cid, pl.ds(sid * rpt, rpt)])

  return sc_agg


def kernel(edge_index, emb, Wl1, Wr1, att1, b1, Wl2, Wr2, att2, b2):
  n = emb.shape[0]
  e2 = edge_index.shape[1] + n
  n_pad = ((n + BN) // BN) * BN
  per_w = -(-e2 // (NW * CHUNK)) * CHUNK
  e_pad = per_w * NW

  loop = jnp.arange(n, dtype=jnp.int32)
  pad_e = e_pad - e2
  src = jnp.concatenate(
      [edge_index[0], loop, jnp.zeros((pad_e,), jnp.int32)])
  dst = jnp.concatenate(
      [edge_index[1], loop, jnp.full((pad_e,), n, jnp.int32)])
  emb_p = jnp.pad(emb, ((0, n_pad - n), (0, 0)))
  zero_nd = jnp.zeros((n_pad, D), jnp.float32)
  b1r = b1.reshape(1, D)
  b2r = b2.reshape(1, D)

  sc_logits = _make_sc_logits(e_pad, n_pad, per_w)
  sc_agg = _make_sc_agg(e_pad, n_pad, per_w)

  xl1, xr1 = _mm2(emb_p, Wl1, Wr1)
  p1, s1 = sc_logits(xl1, xr1, src, dst, att1)
  acc1 = sc_agg(xl1, src, dst, p1, zero_nd)
  h, xl2, xr2 = _comb_mm2(acc1, s1, b1r, emb_p, Wl2, Wr2)
  p2, s2 = sc_logits(xl2, xr2, src, dst, att2)
  acc2 = sc_agg(xl2, src, dst, p2, zero_nd)
  y = _final(acc2, s2, b2r, h)

  y = y[:n]
  return (y[:N_USERS], y[N_USERS:])


# speedup vs baseline: 7.6361x; 7.6361x over previous
import functools

import jax
import jax.numpy as jnp
from jax import lax
from jax.experimental import pallas as pl
from jax.experimental.pallas import tpu as pltpu
from jax.experimental.pallas import tpu_sc as plsc

N_USERS = 6000
D = 128
NC = 2
NS = 16
L = 16
NW = NC * NS
CHUNK = 128
U = 16
BN = 1024


def _mm2_body(x_ref, wl_ref, wr_ref, xl_ref, xr_ref):
  x = x_ref[...]
  xl_ref[...] = jnp.dot(x, wl_ref[...], preferred_element_type=jnp.float32)
  xr_ref[...] = jnp.dot(x, wr_ref[...], preferred_element_type=jnp.float32)


def _mm2(x, wl, wr):
  n = x.shape[0]
  return pl.pallas_call(
      _mm2_body,
      grid=(n // BN,),
      in_specs=[
          pl.BlockSpec((BN, D), lambda i: (i, 0)),
          pl.BlockSpec((D, D), lambda i: (0, 0)),
          pl.BlockSpec((D, D), lambda i: (0, 0)),
      ],
      out_specs=[
          pl.BlockSpec((BN, D), lambda i: (i, 0)),
          pl.BlockSpec((BN, D), lambda i: (i, 0)),
      ],
      out_shape=[jax.ShapeDtypeStruct((n, D), jnp.float32)] * 2,
  )(x, wl, wr)


def _comb_mm2_body(acc_ref, s_ref, b_ref, res_ref, wl_ref, wr_ref,
                   h_ref, xl_ref, xr_ref):
  s = jnp.sum(s_ref[...], axis=0) + 1e-16
  h = (acc_ref[0] + acc_ref[1]) / s[:, None] + b_ref[...] + res_ref[...]
  h = jnp.maximum(h, 0.0)
  h_ref[...] = h
  xl_ref[...] = jnp.dot(h, wl_ref[...], preferred_element_type=jnp.float32)
  xr_ref[...] = jnp.dot(h, wr_ref[...], preferred_element_type=jnp.float32)


def _comb_mm2(acc, s, b, res, wl, wr):
  n = res.shape[0]
  nw = s.shape[0]
  return pl.pallas_call(
      _comb_mm2_body,
      grid=(n // BN,),
      in_specs=[
          pl.BlockSpec((NC, BN, D), lambda i: (0, i, 0)),
          pl.BlockSpec((nw, BN), lambda i: (0, i)),
          pl.BlockSpec((1, D), lambda i: (0, 0)),
          pl.BlockSpec((BN, D), lambda i: (i, 0)),
          pl.BlockSpec((D, D), lambda i: (0, 0)),
          pl.BlockSpec((D, D), lambda i: (0, 0)),
      ],
      out_specs=[
          pl.BlockSpec((BN, D), lambda i: (i, 0)),
          pl.BlockSpec((BN, D), lambda i: (i, 0)),
          pl.BlockSpec((BN, D), lambda i: (i, 0)),
      ],
      out_shape=[jax.ShapeDtypeStruct((n, D), jnp.float32)] * 3,
  )(acc, s, b, res, wl, wr)


def _final_body(acc_ref, s_ref, b_ref, res_ref, y_ref):
  s = jnp.sum(s_ref[...], axis=0) + 1e-16
  y_ref[...] = (acc_ref[0] + acc_ref[1]) / s[:, None] + b_ref[...] + res_ref[...]


def _final(acc, s, b, res):
  n = res.shape[0]
  nw = s.shape[0]
  return pl.pallas_call(
      _final_body,
      grid=(n // BN,),
      in_specs=[
          pl.BlockSpec((NC, BN, D), lambda i: (0, i, 0)),
          pl.BlockSpec((nw, BN), lambda i: (0, i)),
          pl.BlockSpec((1, D), lambda i: (0, 0)),
          pl.BlockSpec((BN, D), lambda i: (i, 0)),
      ],
      out_specs=pl.BlockSpec((BN, D), lambda i: (i, 0)),
      out_shape=jax.ShapeDtypeStruct((n, D), jnp.float32),
  )(acc, s, b, res)


def _sc_mesh():
  return plsc.VectorSubcoreMesh(
      core_axis_name="c", subcore_axis_name="s", num_cores=NC, num_subcores=NS)


_SC_PARAMS = pltpu.CompilerParams(needs_layout_passes=False)


@functools.lru_cache(maxsize=None)
def _make_sc_logits(e_pad, n_pad, per_w):
  n_chunks = per_w // CHUNK

  @functools.partial(
      pl.kernel,
      out_type=[
          jax.ShapeDtypeStruct((e_pad,), jnp.float32),
          jax.ShapeDtypeStruct((NW, n_pad), jnp.float32),
      ],
      mesh=_sc_mesh(),
      compiler_params=_SC_PARAMS,
      scratch_types=[
          pltpu.VMEM((CHUNK,), jnp.int32),
          pltpu.VMEM((CHUNK,), jnp.int32),
          pltpu.VMEM((CHUNK, D), jnp.float32),
          pltpu.VMEM((CHUNK, D), jnp.float32),
          pltpu.VMEM((CHUNK,), jnp.float32),
          pltpu.VMEM((D,), jnp.float32),
          pltpu.VMEM((n_pad,), jnp.float32),
          pltpu.SemaphoreType.DMA,
          pltpu.SemaphoreType.DMA,
      ],
  )
  def sc_logits(xl_hbm, xr_hbm, src_hbm, dst_hbm, att_hbm,
                p_hbm, s_hbm,
                src_v, dst_v, xlr, xrr, p_v, att_v, s_v, sem0, sem1):
    cid = lax.axis_index("c")
    sid = lax.axis_index("s")
    wid = cid * NS + sid
    pltpu.sync_copy(att_hbm, att_v)

    def zero_body(i, carry):
      s_v[pl.ds(i * L, L)] = jnp.zeros((L,), jnp.float32)
      return carry

    lax.fori_loop(0, n_pad // L, zero_body, 0)
    row16 = lax.iota(jnp.int32, L)
    att_q = [att_v[pl.ds(q * L, L)] for q in range(D // L)]

    def chunk_body(i, carry):
      base = wid * per_w + i * CHUNK
      pltpu.sync_copy(src_hbm.at[pl.ds(base, CHUNK)], src_v)
      pltpu.sync_copy(dst_hbm.at[pl.ds(base, CHUNK)], dst_v)
      cl = pltpu.async_copy(xl_hbm.at[src_v], xlr, sem0)
      cr = pltpu.async_copy(xr_hbm.at[dst_v], xrr, sem1)
      cl.wait()
      cr.wait()

      def group_body(g, carry2):
        lvec = jnp.zeros((L,), jnp.float32)
        for u in range(L):
          j = g * L + u
          acc = None
          for q in range(D // L):
            a = xlr[j, pl.ds(q * L, L)]
            b = xrr[j, pl.ds(q * L, L)]
            t = a + b
            t = jnp.maximum(t, 0.2 * t)
            pr = att_q[q] * t
            acc = pr if acc is None else acc + pr
          lvec = jnp.where(row16 == u, jnp.sum(acc), lvec)
        p16 = jnp.exp(lvec)
        p_v[pl.ds(g * L, L)] = p16
        plsc.addupdate_scatter(s_v, [dst_v[pl.ds(g * L, L)]], p16)
        return carry2

      lax.fori_loop(0, CHUNK // L, group_body, 0)
      pltpu.sync_copy(p_v, p_hbm.at[pl.ds(base, CHUNK)])
      return carry

    lax.fori_loop(0, n_chunks, chunk_body, 0)
    pltpu.sync_copy(s_v, s_hbm.at[wid])

  return sc_logits


@functools.lru_cache(maxsize=None)
def _make_sc_agg(e_pad, n_pad, per_w):
  n_chunks = per_w // CHUNK
  rpt = n_pad // NS

  @functools.partial(
      pl.kernel,
      out_type=jax.ShapeDtypeStruct((NC, n_pad, D), jnp.float32),
      mesh=_sc_mesh(),
      compiler_params=_SC_PARAMS,
      scratch_types=[
          pltpu.VMEM((CHUNK,), jnp.int32),
          pltpu.VMEM((CHUNK,), jnp.int32),
          pltpu.VMEM((CHUNK, D), jnp.float32),
          pltpu.VMEM((CHUNK,), jnp.float32),
          pltpu.VMEM_SHARED((n_pad, D), jnp.float32),
          pltpu.SemaphoreType.DMA,
      ],
  )
  def sc_agg(xl_hbm, src_hbm, dst_hbm, p_hbm, zero_nd_hbm,
             out_hbm,
             src_v, dst_v, rows_v, p_v, acc_sh, sem0):
    cid = lax.axis_index("c")
    sid = lax.axis_index("s")
    wid = cid * NS + sid
    pltpu.sync_copy(zero_nd_hbm.at[pl.ds(sid * rpt, rpt)],
                    acc_sh.at[pl.ds(sid * rpt, rpt)])
    plsc.subcore_barrier()

    def chunk_body(i, carry):
      base = wid * per_w + i * CHUNK
      pltpu.sync_copy(src_hbm.at[pl.ds(base, CHUNK)], src_v)
      pltpu.sync_copy(dst_hbm.at[pl.ds(base, CHUNK)], dst_v)
      pltpu.sync_copy(p_hbm.at[pl.ds(base, CHUNK)], p_v)
      pltpu.async_copy(xl_hbm.at[src_v], rows_v, sem0).wait()

      def scale_body(g, carry2):
        wv = p_v[pl.ds(g * L, L)]
        for u in range(L):
          j = g * L + u
          for q in range(D // L):
            sl = pl.ds(q * L, L)
            rows_v[j, sl] = rows_v[j, sl] * wv[u]
        return carry2

      lax.fori_loop(0, CHUNK // L, scale_body, 0)
      pltpu.sync_copy(rows_v, acc_sh.at[dst_v], add=True)
      return carry

    lax.fori_loop(0, n_chunks, chunk_body, 0)
    plsc.subcore_barrier()
    pltpu.sync_copy(acc_sh.at[pl.ds(sid * rpt, rpt)],
                    out_hbm.at[cid, pl.ds(sid * rpt, rpt)])

  return sc_agg


def kernel(edge_index, emb, Wl1, Wr1, att1, b1, Wl2, Wr2, att2, b2):
  n = emb.shape[0]
  e2 = edge_index.shape[1] + n
  n_pad = ((n + BN) // BN) * BN
  per_w = -(-e2 // (NW * CHUNK)) * CHUNK
  e_pad = per_w * NW

  loop = jnp.arange(n, dtype=jnp.int32)
  pad_e = e_pad - e2
  src = jnp.concatenate(
      [edge_index[0], loop, jnp.zeros((pad_e,), jnp.int32)])
  dst = jnp.concatenate(
      [edge_index[1], loop, jnp.full((pad_e,), n, jnp.int32)])
  emb_p = jnp.pad(emb, ((0, n_pad - n), (0, 0)))
  zero_nd = jnp.zeros((n_pad, D), jnp.float32)
  b1r = b1.reshape(1, D)
  b2r = b2.reshape(1, D)

  sc_logits = _make_sc_logits(e_pad, n_pad, per_w)
  sc_agg = _make_sc_agg(e_pad, n_pad, per_w)

  xl1, xr1 = _mm2(emb_p, Wl1, Wr1)
  p1, s1 = sc_logits(xl1, xr1, src, dst, att1)
  acc1 = sc_agg(xl1, src, dst, p1, zero_nd)
  h, xl2, xr2 = _comb_mm2(acc1, s1, b1r, emb_p, Wl2, Wr2)
  p2, s2 = sc_logits(xl2, xr2, src, dst, att2)
  acc2 = sc_agg(xl2, src, dst, p2, zero_nd)
  y = _final(acc2, s2, b2r, h)

  y = y[:n]
  return (y[:N_USERS], y[N_USERS:])


# speedup vs baseline: 12.0318x; 1.5756x over previous
import functools

import jax
import jax.numpy as jnp
from jax import lax
from jax.experimental import pallas as pl
from jax.experimental.pallas import tpu as pltpu
from jax.experimental.pallas import tpu_sc as plsc

N_USERS = 6000
D = 128
NC = 2
NS = 16
L = 16
NW = NC * NS
CHUNK = 128
U = 16
BN = 1024


def _mm2_body(x_ref, wl_ref, wr_ref, xl_ref, xr_ref):
  x = x_ref[...]
  xl_ref[...] = jnp.dot(x, wl_ref[...], preferred_element_type=jnp.float32)
  xr_ref[...] = jnp.dot(x, wr_ref[...], preferred_element_type=jnp.float32)


def _mm2(x, wl, wr):
  n = x.shape[0]
  return pl.pallas_call(
      _mm2_body,
      grid=(n // BN,),
      in_specs=[
          pl.BlockSpec((BN, D), lambda i: (i, 0)),
          pl.BlockSpec((D, D), lambda i: (0, 0)),
          pl.BlockSpec((D, D), lambda i: (0, 0)),
      ],
      out_specs=[
          pl.BlockSpec((BN, D), lambda i: (i, 0)),
          pl.BlockSpec((BN, D), lambda i: (i, 0)),
      ],
      out_shape=[jax.ShapeDtypeStruct((n, D), jnp.float32)] * 2,
  )(x, wl, wr)


def _comb_mm2_body(acc_ref, s_ref, b_ref, res_ref, wl_ref, wr_ref,
                   h_ref, xl_ref, xr_ref):
  s = jnp.sum(s_ref[...], axis=0) + 1e-16
  h = (acc_ref[0] + acc_ref[1]) / s[:, None] + b_ref[...] + res_ref[...]
  h = jnp.maximum(h, 0.0)
  h_ref[...] = h
  xl_ref[...] = jnp.dot(h, wl_ref[...], preferred_element_type=jnp.float32)
  xr_ref[...] = jnp.dot(h, wr_ref[...], preferred_element_type=jnp.float32)


def _comb_mm2(acc, s, b, res, wl, wr):
  n = res.shape[0]
  nw = s.shape[0]
  return pl.pallas_call(
      _comb_mm2_body,
      grid=(n // BN,),
      in_specs=[
          pl.BlockSpec((NC, BN, D), lambda i: (0, i, 0)),
          pl.BlockSpec((nw, BN), lambda i: (0, i)),
          pl.BlockSpec((1, D), lambda i: (0, 0)),
          pl.BlockSpec((BN, D), lambda i: (i, 0)),
          pl.BlockSpec((D, D), lambda i: (0, 0)),
          pl.BlockSpec((D, D), lambda i: (0, 0)),
      ],
      out_specs=[
          pl.BlockSpec((BN, D), lambda i: (i, 0)),
          pl.BlockSpec((BN, D), lambda i: (i, 0)),
          pl.BlockSpec((BN, D), lambda i: (i, 0)),
      ],
      out_shape=[jax.ShapeDtypeStruct((n, D), jnp.float32)] * 3,
  )(acc, s, b, res, wl, wr)


def _final_body(acc_ref, s_ref, b_ref, res_ref, y_ref):
  s = jnp.sum(s_ref[...], axis=0) + 1e-16
  y_ref[...] = (acc_ref[0] + acc_ref[1]) / s[:, None] + b_ref[...] + res_ref[...]


def _final(acc, s, b, res):
  n = res.shape[0]
  nw = s.shape[0]
  return pl.pallas_call(
      _final_body,
      grid=(n // BN,),
      in_specs=[
          pl.BlockSpec((NC, BN, D), lambda i: (0, i, 0)),
          pl.BlockSpec((nw, BN), lambda i: (0, i)),
          pl.BlockSpec((1, D), lambda i: (0, 0)),
          pl.BlockSpec((BN, D), lambda i: (i, 0)),
      ],
      out_specs=pl.BlockSpec((BN, D), lambda i: (i, 0)),
      out_shape=jax.ShapeDtypeStruct((n, D), jnp.float32),
  )(acc, s, b, res)


def _sc_mesh():
  return plsc.VectorSubcoreMesh(
      core_axis_name="c", subcore_axis_name="s", num_cores=NC, num_subcores=NS)


_SC_PARAMS = pltpu.CompilerParams(needs_layout_passes=False)


@functools.lru_cache(maxsize=None)
def _make_sc_edge(e_pad, n_pad, per_w):
  n_chunks = per_w // CHUNK
  rpt = n_pad // NS

  @functools.partial(
      pl.kernel,
      out_type=[
          jax.ShapeDtypeStruct((NW, n_pad), jnp.float32),
          jax.ShapeDtypeStruct((NC, n_pad, D), jnp.float32),
      ],
      mesh=_sc_mesh(),
      compiler_params=_SC_PARAMS,
      scratch_types=[
          pltpu.VMEM((CHUNK,), jnp.int32),
          pltpu.VMEM((CHUNK,), jnp.int32),
          pltpu.VMEM((CHUNK, D), jnp.float32),
          pltpu.VMEM((CHUNK, D), jnp.float32),
          pltpu.VMEM((D,), jnp.float32),
          pltpu.VMEM((n_pad,), jnp.float32),
          pltpu.VMEM_SHARED((n_pad, D), jnp.float32),
          pltpu.SemaphoreType.DMA,
          pltpu.SemaphoreType.DMA,
      ],
  )
  def sc_edge(xl_hbm, xr_hbm, src_hbm, dst_hbm, att_hbm, zero_nd_hbm,
              s_hbm, out_hbm,
              src_v, dst_v, xlr, xrr, att_v, s_v, acc_sh, sem0, sem1):
    cid = lax.axis_index("c")
    sid = lax.axis_index("s")
    wid = cid * NS + sid
    pltpu.sync_copy(att_hbm, att_v)
    pltpu.sync_copy(zero_nd_hbm.at[pl.ds(sid * rpt, rpt)],
                    acc_sh.at[pl.ds(sid * rpt, rpt)])

    def zero_body(i, carry):
      s_v[pl.ds(i * L, L)] = jnp.zeros((L,), jnp.float32)
      return carry

    lax.fori_loop(0, n_pad // L, zero_body, 0)
    plsc.subcore_barrier()
    row16 = lax.iota(jnp.int32, L)
    att_q = [att_v[pl.ds(q * L, L)] for q in range(D // L)]

    def chunk_body(i, carry):
      base = wid * per_w + i * CHUNK
      pltpu.sync_copy(src_hbm.at[pl.ds(base, CHUNK)], src_v)
      pltpu.sync_copy(dst_hbm.at[pl.ds(base, CHUNK)], dst_v)
      cl = pltpu.async_copy(xl_hbm.at[src_v], xlr, sem0)
      cr = pltpu.async_copy(xr_hbm.at[dst_v], xrr, sem1)
      cl.wait()
      cr.wait()

      def group_body(g, carry2):
        lvec = jnp.zeros((L,), jnp.float32)
        for u in range(L):
          j = g * L + u
          acc = None
          for q in range(D // L):
            a = xlr[j, pl.ds(q * L, L)]
            b = xrr[j, pl.ds(q * L, L)]
            t = a + b
            t = jnp.maximum(t, 0.2 * t)
            pr = att_q[q] * t
            acc = pr if acc is None else acc + pr
          lvec = jnp.where(row16 == u, jnp.sum(acc), lvec)
        p16 = jnp.exp(lvec)
        plsc.addupdate_scatter(s_v, [dst_v[pl.ds(g * L, L)]], p16)
        for u in range(L):
          j = g * L + u
          for q in range(D // L):
            sl = pl.ds(q * L, L)
            xlr[j, sl] = xlr[j, sl] * p16[u]
        return carry2

      lax.fori_loop(0, CHUNK // L, group_body, 0)
      pltpu.sync_copy(xlr, acc_sh.at[dst_v], add=True)
      return carry

    lax.fori_loop(0, n_chunks, chunk_body, 0)
    pltpu.sync_copy(s_v, s_hbm.at[wid])
    plsc.subcore_barrier()
    pltpu.sync_copy(acc_sh.at[pl.ds(sid * rpt, rpt)],
                    out_hbm.at[cid, pl.ds(sid * rpt, rpt)])

  return sc_edge


def kernel(edge_index, emb, Wl1, Wr1, att1, b1, Wl2, Wr2, att2, b2):
  n = emb.shape[0]
  e2 = edge_index.shape[1] + n
  n_pad = ((n + BN) // BN) * BN
  per_w = -(-e2 // (NW * CHUNK)) * CHUNK
  e_pad = per_w * NW

  loop = jnp.arange(n, dtype=jnp.int32)
  pad_e = e_pad - e2
  src = jnp.concatenate(
      [edge_index[0], loop, jnp.zeros((pad_e,), jnp.int32)])
  dst = jnp.concatenate(
      [edge_index[1], loop, jnp.full((pad_e,), n, jnp.int32)])
  emb_p = jnp.pad(emb, ((0, n_pad - n), (0, 0)))
  zero_nd = jnp.zeros((n_pad, D), jnp.float32)
  b1r = b1.reshape(1, D)
  b2r = b2.reshape(1, D)

  sc_edge = _make_sc_edge(e_pad, n_pad, per_w)

  xl1, xr1 = _mm2(emb_p, Wl1, Wr1)
  s1, acc1 = sc_edge(xl1, xr1, src, dst, att1, zero_nd)
  h, xl2, xr2 = _comb_mm2(acc1, s1, b1r, emb_p, Wl2, Wr2)
  s2, acc2 = sc_edge(xl2, xr2, src, dst, att2, zero_nd)
  y = _final(acc2, s2, b2r, h)

  y = y[:n]
  return (y[:N_USERS], y[N_USERS:])


# speedup vs baseline: 14.9631x; 1.2436x over previous
import functools

import jax
import jax.numpy as jnp
from jax import lax
from jax.experimental import pallas as pl
from jax.experimental.pallas import tpu as pltpu
from jax.experimental.pallas import tpu_sc as plsc

N_USERS = 6000
D = 128
NC = 2
NS = 16
L = 16
NW = NC * NS
CHUNK = 64
U = 16
BN = 1024


def _mm2_body(x_ref, wl_ref, wr_ref, xl_ref, xr_ref):
  x = x_ref[...]
  xl_ref[...] = jnp.dot(x, wl_ref[...], preferred_element_type=jnp.float32)
  xr_ref[...] = jnp.dot(x, wr_ref[...], preferred_element_type=jnp.float32)


def _mm2(x, wl, wr):
  n = x.shape[0]
  return pl.pallas_call(
      _mm2_body,
      grid=(n // BN,),
      in_specs=[
          pl.BlockSpec((BN, D), lambda i: (i, 0)),
          pl.BlockSpec((D, D), lambda i: (0, 0)),
          pl.BlockSpec((D, D), lambda i: (0, 0)),
      ],
      out_specs=[
          pl.BlockSpec((BN, D), lambda i: (i, 0)),
          pl.BlockSpec((BN, D), lambda i: (i, 0)),
      ],
      out_shape=[jax.ShapeDtypeStruct((n, D), jnp.float32)] * 2,
  )(x, wl, wr)


def _comb_mm2_body(acc_ref, s_ref, b_ref, res_ref, wl_ref, wr_ref,
                   h_ref, xl_ref, xr_ref):
  s = jnp.sum(s_ref[...], axis=0) + 1e-16
  h = (acc_ref[0] + acc_ref[1]) / s[:, None] + b_ref[...] + res_ref[...]
  h = jnp.maximum(h, 0.0)
  h_ref[...] = h
  xl_ref[...] = jnp.dot(h, wl_ref[...], preferred_element_type=jnp.float32)
  xr_ref[...] = jnp.dot(h, wr_ref[...], preferred_element_type=jnp.float32)


def _comb_mm2(acc, s, b, res, wl, wr):
  n = res.shape[0]
  nw = s.shape[0]
  return pl.pallas_call(
      _comb_mm2_body,
      grid=(n // BN,),
      in_specs=[
          pl.BlockSpec((NC, BN, D), lambda i: (0, i, 0)),
          pl.BlockSpec((nw, BN), lambda i: (0, i)),
          pl.BlockSpec((1, D), lambda i: (0, 0)),
          pl.BlockSpec((BN, D), lambda i: (i, 0)),
          pl.BlockSpec((D, D), lambda i: (0, 0)),
          pl.BlockSpec((D, D), lambda i: (0, 0)),
      ],
      out_specs=[
          pl.BlockSpec((BN, D), lambda i: (i, 0)),
          pl.BlockSpec((BN, D), lambda i: (i, 0)),
          pl.BlockSpec((BN, D), lambda i: (i, 0)),
      ],
      out_shape=[jax.ShapeDtypeStruct((n, D), jnp.float32)] * 3,
  )(acc, s, b, res, wl, wr)


def _final_body(acc_ref, s_ref, b_ref, res_ref, y_ref):
  s = jnp.sum(s_ref[...], axis=0) + 1e-16
  y_ref[...] = (acc_ref[0] + acc_ref[1]) / s[:, None] + b_ref[...] + res_ref[...]


def _final(acc, s, b, res):
  n = res.shape[0]
  nw = s.shape[0]
  return pl.pallas_call(
      _final_body,
      grid=(n // BN,),
      in_specs=[
          pl.BlockSpec((NC, BN, D), lambda i: (0, i, 0)),
          pl.BlockSpec((nw, BN), lambda i: (0, i)),
          pl.BlockSpec((1, D), lambda i: (0, 0)),
          pl.BlockSpec((BN, D), lambda i: (i, 0)),
      ],
      out_specs=pl.BlockSpec((BN, D), lambda i: (i, 0)),
      out_shape=jax.ShapeDtypeStruct((n, D), jnp.float32),
  )(acc, s, b, res)


def _sc_mesh():
  return plsc.VectorSubcoreMesh(
      core_axis_name="c", subcore_axis_name="s", num_cores=NC, num_subcores=NS)


_SC_PARAMS = pltpu.CompilerParams(needs_layout_passes=False)


@functools.lru_cache(maxsize=None)
def _make_sc_edge(e_pad, n_pad, per_w):
  n_chunks = per_w // CHUNK
  n_pairs = n_chunks // 2
  rpt = n_pad // NS

  @functools.partial(
      pl.kernel,
      out_type=[
          jax.ShapeDtypeStruct((NW, n_pad), jnp.float32),
          jax.ShapeDtypeStruct((NC, n_pad, D), jnp.float32),
      ],
      mesh=_sc_mesh(),
      compiler_params=_SC_PARAMS,
      scratch_types=[
          pltpu.VMEM((CHUNK,), jnp.int32),
          pltpu.VMEM((CHUNK,), jnp.int32),
          pltpu.VMEM((CHUNK, D), jnp.float32),
          pltpu.VMEM((CHUNK, D), jnp.float32),
          pltpu.VMEM((CHUNK,), jnp.int32),
          pltpu.VMEM((CHUNK,), jnp.int32),
          pltpu.VMEM((CHUNK, D), jnp.float32),
          pltpu.VMEM((CHUNK, D), jnp.float32),
          pltpu.VMEM((D,), jnp.float32),
          pltpu.VMEM((n_pad,), jnp.float32),
          pltpu.VMEM_SHARED((n_pad, D), jnp.float32),
          pltpu.SemaphoreType.DMA,
          pltpu.SemaphoreType.DMA,
      ],
  )
  def sc_edge(xl_hbm, xr_hbm, src_hbm, dst_hbm, att_hbm, zero_nd_hbm,
              s_hbm, out_hbm,
              src_a, dst_a, xlr_a, xrr_a, src_b, dst_b, xlr_b, xrr_b,
              att_v, s_v, acc_sh, sem_a, sem_b):
    cid = lax.axis_index("c")
    sid = lax.axis_index("s")
    wid = cid * NS + sid
    pltpu.sync_copy(att_hbm, att_v)
    pltpu.sync_copy(zero_nd_hbm.at[pl.ds(sid * rpt, rpt)],
                    acc_sh.at[pl.ds(sid * rpt, rpt)])

    def zero_body(i, carry):
      s_v[pl.ds(i * L, L)] = jnp.zeros((L,), jnp.float32)
      return carry

    lax.fori_loop(0, n_pad // L, zero_body, 0)
    plsc.subcore_barrier()
    row16 = lax.iota(jnp.int32, L)
    att_q = [att_v[pl.ds(q * L, L)] for q in range(D // L)]

    def issue(ci, srcb, dstb, xlb, xrb, sem):
      base = wid * per_w + ci * CHUNK
      pltpu.sync_copy(src_hbm.at[pl.ds(base, CHUNK)], srcb)
      pltpu.sync_copy(dst_hbm.at[pl.ds(base, CHUNK)], dstb)
      pltpu.async_copy(xl_hbm.at[srcb], xlb, sem)
      pltpu.async_copy(xr_hbm.at[dstb], xrb, sem)

    def wait(srcb, dstb, xlb, xrb, sem):
      pltpu.make_async_copy(xl_hbm.at[srcb], xlb, sem).wait()
      pltpu.make_async_copy(xr_hbm.at[dstb], xrb, sem).wait()

    def process(dstb, xlb, xrb):
      def group_body(g, carry2):
        lvec = jnp.zeros((L,), jnp.float32)
        for u in range(L):
          j = g * L + u
          acc = None
          for q in range(D // L):
            a = xlb[j, pl.ds(q * L, L)]
            b = xrb[j, pl.ds(q * L, L)]
            t = a + b
            t = jnp.maximum(t, 0.2 * t)
            pr = att_q[q] * t
            acc = pr if acc is None else acc + pr
          lvec = jnp.where(row16 == u, jnp.sum(acc), lvec)
        p16 = jnp.exp(lvec)
        plsc.addupdate_scatter(s_v, [dst_v_slice(dstb, g)], p16)
        for u in range(L):
          j = g * L + u
          for q in range(D // L):
            sl = pl.ds(q * L, L)
            xlb[j, sl] = xlb[j, sl] * p16[u]
        return carry2

      lax.fori_loop(0, CHUNK // L, group_body, 0)
      pltpu.sync_copy(xlb, acc_sh.at[dstb], add=True)

    def dst_v_slice(dstb, g):
      return dstb[pl.ds(g * L, L)]

    issue(0, src_a, dst_a, xlr_a, xrr_a, sem_a)

    def pair_body(i, carry):
      issue(2 * i + 1, src_b, dst_b, xlr_b, xrr_b, sem_b)
      wait(src_a, dst_a, xlr_a, xrr_a, sem_a)
      process(dst_a, xlr_a, xrr_a)

      @pl.when(i < n_pairs - 1)
      def _():
        issue(2 * i + 2, src_a, dst_a, xlr_a, xrr_a, sem_a)

      wait(src_b, dst_b, xlr_b, xrr_b, sem_b)
      process(dst_b, xlr_b, xrr_b)
      return carry

    lax.fori_loop(0, n_pairs, pair_body, 0)
    pltpu.sync_copy(s_v, s_hbm.at[wid])
    plsc.subcore_barrier()
    pltpu.sync_copy(acc_sh.at[pl.ds(sid * rpt, rpt)],
                    out_hbm.at[cid, pl.ds(sid * rpt, rpt)])

  return sc_edge


def kernel(edge_index, emb, Wl1, Wr1, att1, b1, Wl2, Wr2, att2, b2):
  n = emb.shape[0]
  e2 = edge_index.shape[1] + n
  n_pad = ((n + BN) // BN) * BN
  per_w = -(-e2 // (NW * 2 * CHUNK)) * 2 * CHUNK
  e_pad = per_w * NW

  loop = jnp.arange(n, dtype=jnp.int32)
  pad_e = e_pad - e2
  src = jnp.concatenate(
      [edge_index[0], loop, jnp.zeros((pad_e,), jnp.int32)])
  dst = jnp.concatenate(
      [edge_index[1], loop, jnp.full((pad_e,), n, jnp.int32)])
  emb_p = jnp.pad(emb, ((0, n_pad - n), (0, 0)))
  zero_nd = jnp.zeros((n_pad, D), jnp.float32)
  b1r = b1.reshape(1, D)
  b2r = b2.reshape(1, D)

  sc_edge = _make_sc_edge(e_pad, n_pad, per_w)

  xl1, xr1 = _mm2(emb_p, Wl1, Wr1)
  s1, acc1 = sc_edge(xl1, xr1, src, dst, att1, zero_nd)
  h, xl2, xr2 = _comb_mm2(acc1, s1, b1r, emb_p, Wl2, Wr2)
  s2, acc2 = sc_edge(xl2, xr2, src, dst, att2, zero_nd)
  y = _final(acc2, s2, b2r, h)

  y = y[:n]
  return (y[:N_USERS], y[N_USERS:])
